# Initial kernel scaffold; baseline (speedup 1.0000x reference)
import jax
import jax.numpy as jnp
from jax.experimental import pallas as pl


def kernel(v0, a0, v, a, av, W, b):
    raise NotImplementedError("write your pallas kernel here")


# speedup vs baseline: 1.9885x; 1.9885x over previous
import jax
import jax.numpy as jnp
from jax.experimental import pallas as pl
from jax.experimental.pallas import tpu as pltpu

_B, _S, _D, _E = 4, 2048, 1024, 3
_S_BLK = 512
_S_BLKS = _S // _S_BLK


def _stats_body(v0_ref, a0_ref, w_ref, bias_ref, logits_ref, acc_ref):
    j = pl.program_id(1)

    @pl.when(j == 0)
    def _():
        acc_ref[...] = jnp.zeros_like(acc_ref)

    vb = v0_ref[0]
    ab = a0_ref[0]
    acc_ref[0, :] += jnp.sum(vb, axis=0)
    acc_ref[1, :] += jnp.sum(vb * vb, axis=0)
    acc_ref[2, :] += jnp.sum(ab, axis=0)
    acc_ref[3, :] += jnp.sum(ab * ab, axis=0)

    @pl.when(j == _S_BLKS - 1)
    def _():
        inv_s = 1.0 / _S
        inv_n1 = 1.0 / (_S - 1)
        mean_v = acc_ref[0:1, :] * inv_s
        var_v = (acc_ref[1:2, :] - _S * mean_v * mean_v) * inv_n1
        mean_a = acc_ref[2:3, :] * inv_s
        var_a = (acc_ref[3:4, :] - _S * mean_a * mean_a) * inv_n1
        feats = jnp.concatenate(
            [mean_v, jnp.sqrt(var_v), mean_a, jnp.sqrt(var_a)], axis=1
        )
        logits = jnp.sum(w_ref[...] * feats, axis=1) + bias_ref[0]
        logits_ref[0, 0, :] = logits


def _argmax3(lg_ref, b):
    l0 = lg_ref[3 * b]
    l1 = lg_ref[3 * b + 1]
    l2 = lg_ref[3 * b + 2]
    i01 = jnp.where(l1 > l0, 1, 0)
    m01 = jnp.maximum(l0, l1)
    return jnp.where(l2 > m01, 2, i01)


def _expert_index_map(e):
    def index_map(b, j, lg_ref):
        sel = _argmax3(lg_ref, b) == e
        return (jnp.where(sel, b, 0), jnp.where(sel, j, 0), 0)

    return index_map


def _route_body(lg_ref, v_ref, a_ref, av_ref, o_ref):
    e = _argmax3(lg_ref, pl.program_id(0))

    @pl.when(e == 0)
    def _():
        o_ref[...] = v_ref[...]

    @pl.when(e == 1)
    def _():
        o_ref[...] = a_ref[...]

    @pl.when(e == 2)
    def _():
        o_ref[...] = av_ref[...]


def kernel(v0, a0, v, a, av, W, b):
    logits3 = pl.pallas_call(
        _stats_body,
        grid=(_B, _S_BLKS),
        in_specs=[
            pl.BlockSpec((1, _S_BLK, _D), lambda bi, j: (bi, j, 0)),
            pl.BlockSpec((1, _S_BLK, _D), lambda bi, j: (bi, j, 0)),
            pl.BlockSpec((_E, 4 * _D), lambda bi, j: (0, 0)),
            pl.BlockSpec((1, _E), lambda bi, j: (0, 0)),
        ],
        out_specs=pl.BlockSpec((1, 1, _E), lambda bi, j: (bi, 0, 0)),
        out_shape=jax.ShapeDtypeStruct((_B, 1, _E), jnp.float32),
        scratch_shapes=[pltpu.VMEM((8, _D), jnp.float32)],
        compiler_params=pltpu.CompilerParams(
            dimension_semantics=("parallel", "arbitrary")
        ),
    )(v0, a0, W, b.reshape(1, _E))

    logits = logits3.reshape(_B, _E)

    combined = pl.pallas_call(
        _route_body,
        grid_spec=pltpu.PrefetchScalarGridSpec(
            num_scalar_prefetch=1,
            grid=(_B, _S_BLKS),
            in_specs=[
                pl.BlockSpec((1, _S_BLK, _D), _expert_index_map(0)),
                pl.BlockSpec((1, _S_BLK, _D), _expert_index_map(1)),
                pl.BlockSpec((1, _S_BLK, _D), _expert_index_map(2)),
            ],
            out_specs=pl.BlockSpec((1, _S_BLK, _D), lambda bi, j, lg: (bi, j, 0)),
        ),
        out_shape=jax.ShapeDtypeStruct((_B, _S, _D), jnp.float32),
        compiler_params=pltpu.CompilerParams(
            dimension_semantics=("parallel", "arbitrary")
        ),
    )(logits.reshape(_B * _E), v, a, av)

    return combined, logits


# speedup vs baseline: 2.1615x; 1.0870x over previous
import jax
import jax.numpy as jnp
from jax.experimental import pallas as pl
from jax.experimental.pallas import tpu as pltpu

_B, _S, _D, _E = 4, 2048, 1024, 3
_S_BLK = 1024
_S_BLKS = _S // _S_BLK


def _stats_body(v0_ref, a0_ref, w_ref, bias_ref, logits_ref, acc_ref):
    j = pl.program_id(1)

    @pl.when(j == 0)
    def _():
        acc_ref[...] = jnp.zeros_like(acc_ref)

    vb = v0_ref[0]
    ab = a0_ref[0]
    acc_ref[0, :] += jnp.sum(vb, axis=0)
    acc_ref[1, :] += jnp.sum(vb * vb, axis=0)
    acc_ref[2, :] += jnp.sum(ab, axis=0)
    acc_ref[3, :] += jnp.sum(ab * ab, axis=0)

    @pl.when(j == _S_BLKS - 1)
    def _():
        inv_s = 1.0 / _S
        inv_n1 = 1.0 / (_S - 1)
        mean_v = acc_ref[0:1, :] * inv_s
        var_v = (acc_ref[1:2, :] - _S * mean_v * mean_v) * inv_n1
        mean_a = acc_ref[2:3, :] * inv_s
        var_a = (acc_ref[3:4, :] - _S * mean_a * mean_a) * inv_n1
        feats = jnp.concatenate(
            [mean_v, jnp.sqrt(var_v), mean_a, jnp.sqrt(var_a)], axis=1
        )
        logits = jnp.sum(w_ref[...] * feats, axis=1) + bias_ref[0]
        logits_ref[0, 0, :] = logits


def _argmax3(lg_ref, b):
    l0 = lg_ref[3 * b]
    l1 = lg_ref[3 * b + 1]
    l2 = lg_ref[3 * b + 2]
    i01 = jnp.where(l1 > l0, 1, 0)
    m01 = jnp.maximum(l0, l1)
    return jnp.where(l2 > m01, 2, i01)


def _expert_index_map(e):
    def index_map(b, j, lg_ref):
        sel = _argmax3(lg_ref, b) == e
        return (jnp.where(sel, b, 0), jnp.where(sel, j, 0), 0)

    return index_map


def _route_body(lg_ref, v_ref, a_ref, av_ref, o_ref):
    e = _argmax3(lg_ref, pl.program_id(0))

    @pl.when(e == 0)
    def _():
        o_ref[...] = v_ref[...]

    @pl.when(e == 1)
    def _():
        o_ref[...] = a_ref[...]

    @pl.when(e == 2)
    def _():
        o_ref[...] = av_ref[...]


def kernel(v0, a0, v, a, av, W, b):
    logits3 = pl.pallas_call(
        _stats_body,
        grid=(_B, _S_BLKS),
        in_specs=[
            pl.BlockSpec((1, _S_BLK, _D), lambda bi, j: (bi, j, 0)),
            pl.BlockSpec((1, _S_BLK, _D), lambda bi, j: (bi, j, 0)),
            pl.BlockSpec((_E, 4 * _D), lambda bi, j: (0, 0)),
            pl.BlockSpec((1, _E), lambda bi, j: (0, 0)),
        ],
        out_specs=pl.BlockSpec((1, 1, _E), lambda bi, j: (bi, 0, 0)),
        out_shape=jax.ShapeDtypeStruct((_B, 1, _E), jnp.float32),
        scratch_shapes=[pltpu.VMEM((8, _D), jnp.float32)],
        compiler_params=pltpu.CompilerParams(
            dimension_semantics=("parallel", "arbitrary")
        ),
    )(v0, a0, W, b.reshape(1, _E))

    logits = logits3.reshape(_B, _E)

    combined = pl.pallas_call(
        _route_body,
        grid_spec=pltpu.PrefetchScalarGridSpec(
            num_scalar_prefetch=1,
            grid=(_B, _S_BLKS),
            in_specs=[
                pl.BlockSpec((1, _S_BLK, _D), _expert_index_map(0)),
                pl.BlockSpec((1, _S_BLK, _D), _expert_index_map(1)),
                pl.BlockSpec((1, _S_BLK, _D), _expert_index_map(2)),
            ],
            out_specs=pl.BlockSpec((1, _S_BLK, _D), lambda bi, j, lg: (bi, j, 0)),
        ),
        out_shape=jax.ShapeDtypeStruct((_B, _S, _D), jnp.float32),
        compiler_params=pltpu.CompilerParams(
            dimension_semantics=("parallel", "arbitrary")
        ),
    )(logits.reshape(_B * _E), v, a, av)

    return combined, logits
